# Initial kernel scaffold; baseline (speedup 1.0000x reference)
import jax
import jax.numpy as jnp
from jax.experimental import pallas as pl


def kernel(edge_index, user_emb, item_emb):
    raise NotImplementedError("write your pallas kernel here")


# speedup vs baseline: 14.1782x; 14.1782x over previous
import functools

import jax
import jax.numpy as jnp
from jax import lax
from jax.experimental import pallas as pl
from jax.experimental.pallas import tpu as pltpu
from jax.experimental.pallas import tpu_sc as plsc

N_USER = 25000
N_SHOP = 25000
N_TOTAL = 50000
E = 800000
D = 64
N_LAYERS = 3

NC = 2
NS = 16
IW = 128
ROWS = E // IW
RPT = -(-ROWS // NS)
NPAD = 25088
ZCH = NPAD // NS
DEGW = 8

OCH = 1000
ONCH = N_USER // OCH


def _mesh():
  return plsc.VectorSubcoreMesh(
      core_axis_name="c", subcore_axis_name="s", num_cores=NC, num_subcores=NS
  )


def _deg_body(ssrc, zpat, opat, deg_out, idx_s, obuf, deg_sh):
  c = lax.axis_index("c")
  s = lax.axis_index("s")
  pltpu.sync_copy(zpat, deg_sh.at[pl.ds(s * ZCH, ZCH)])
  pltpu.sync_copy(opat, obuf)
  plsc.subcore_barrier()

  def body(k, carry):
    row = k * NS + s

    @pl.when(row < ROWS)
    def _():
      pltpu.sync_copy(ssrc.at[c, row], idx_s)
      pltpu.sync_copy(obuf, deg_sh.at[idx_s], add=True)

    return carry

  lax.fori_loop(0, RPT, body, 0)
  plsc.subcore_barrier()

  def out_body(rep, carry):
    k = rep * NS + s

    @pl.when(k < ONCH)
    def _():
      pltpu.sync_copy(
          deg_sh.at[pl.ds(k * OCH, OCH)],
          deg_out.at[pl.ds(c * N_USER + k * OCH, OCH)],
      )

    return carry

  lax.fori_loop(0, 2, out_body, 0)


_sc_params = pltpu.CompilerParams(use_tc_tiling_on_sc=False)

_deg_call = pl.kernel(
    _deg_body,
    out_type=jax.ShapeDtypeStruct((N_TOTAL, DEGW), jnp.float32),
    mesh=_mesh(),
    compiler_params=_sc_params,
    scratch_types=[
        pltpu.VMEM((IW,), jnp.int32),
        pltpu.VMEM((IW, DEGW), jnp.float32),
        pltpu.VMEM_SHARED((NPAD, DEGW), jnp.float32),
    ],
)


def _spmm_body(z, gsrc, ssrc, zpat64, acc_out, idx_g, idx_s, rows_v, acc_sh, sem):
  c = lax.axis_index("c")
  s = lax.axis_index("s")
  pltpu.sync_copy(zpat64, acc_sh.at[pl.ds(s * ZCH, ZCH)])
  plsc.subcore_barrier()

  def body(k, carry):
    row = k * NS + s

    @pl.when(row < ROWS)
    def _():
      pltpu.sync_copy(gsrc.at[c, row], idx_g)
      pltpu.sync_copy(ssrc.at[c, row], idx_s)
      pltpu.async_copy(z.at[idx_g], rows_v, sem).wait()
      pltpu.sync_copy(rows_v, acc_sh.at[idx_s], add=True)

    return carry

  lax.fori_loop(0, RPT, body, 0)
  plsc.subcore_barrier()

  def out_body(rep, carry):
    k = rep * NS + s

    @pl.when(k < ONCH)
    def _():
      pltpu.sync_copy(
          acc_sh.at[pl.ds(k * OCH, OCH)],
          acc_out.at[pl.ds(c * N_USER + k * OCH, OCH)],
      )

    return carry

  lax.fori_loop(0, 2, out_body, 0)


_spmm_call = pl.kernel(
    _spmm_body,
    out_type=jax.ShapeDtypeStruct((N_TOTAL, D), jnp.float32),
    mesh=_mesh(),
    compiler_params=_sc_params,
    scratch_types=[
        pltpu.VMEM((IW,), jnp.int32),
        pltpu.VMEM((IW,), jnp.int32),
        pltpu.VMEM((IW, D), jnp.float32),
        pltpu.VMEM_SHARED((NPAD, D), jnp.float32),
        pltpu.SemaphoreType.DMA,
    ],
)


BR = 2000
GR = N_TOTAL // BR


def _scale0_body(deg_ref, emb_ref, dinv_ref, z_ref):
  deg = deg_ref[...][:, 0:1]
  dinv = jnp.where(deg > 0, lax.rsqrt(jnp.maximum(deg, 1e-12)), 0.0)
  dinv_ref[...] = dinv
  z_ref[...] = emb_ref[...] * dinv


_scale0_call = pl.pallas_call(
    _scale0_body,
    grid=(GR,),
    in_specs=[
        pl.BlockSpec((BR, DEGW), lambda i: (i, 0)),
        pl.BlockSpec((BR, D), lambda i: (i, 0)),
    ],
    out_specs=[
        pl.BlockSpec((BR, 1), lambda i: (i, 0)),
        pl.BlockSpec((BR, D), lambda i: (i, 0)),
    ],
    out_shape=[
        jax.ShapeDtypeStruct((N_TOTAL, 1), jnp.float32),
        jax.ShapeDtypeStruct((N_TOTAL, D), jnp.float32),
    ],
)


def _scale_body(acc_ref, dinv_ref, s_ref, sout_ref, z_ref, *, final):
  dinv = dinv_ref[...]
  x = acc_ref[...] * dinv
  sn = s_ref[...] + x
  sout_ref[...] = sn * 0.25 if final else sn
  z_ref[...] = x * dinv


def _make_scale(final):
  return pl.pallas_call(
      functools.partial(_scale_body, final=final),
      grid=(GR,),
      in_specs=[
          pl.BlockSpec((BR, D), lambda i: (i, 0)),
          pl.BlockSpec((BR, 1), lambda i: (i, 0)),
          pl.BlockSpec((BR, D), lambda i: (i, 0)),
      ],
      out_specs=[
          pl.BlockSpec((BR, D), lambda i: (i, 0)),
          pl.BlockSpec((BR, D), lambda i: (i, 0)),
      ],
      out_shape=[
          jax.ShapeDtypeStruct((N_TOTAL, D), jnp.float32),
          jax.ShapeDtypeStruct((N_TOTAL, D), jnp.float32),
      ],
  )


_scale_mid = _make_scale(False)
_scale_fin = _make_scale(True)


@jax.jit
def kernel(edge_index, user_emb, item_emb):
  users = edge_index[0].astype(jnp.int32)
  items = edge_index[1].astype(jnp.int32)
  u2 = users.reshape(ROWS, IW)
  i2 = items.reshape(ROWS, IW)
  gsrc = jnp.stack([i2, u2])
  ssrc = jnp.stack([u2, i2 - N_USER])
  zpat = jnp.zeros((ZCH, DEGW), jnp.float32)
  opat = jnp.tile(jnp.eye(1, DEGW, dtype=jnp.float32), (IW, 1))
  zpat64 = jnp.zeros((ZCH, D), jnp.float32)
  emb = jnp.concatenate([user_emb, item_emb], axis=0)

  deg8 = _deg_call(ssrc, zpat, opat)
  dinv, zcur = _scale0_call(deg8, emb)
  scur = emb
  for l in range(N_LAYERS):
    acc = _spmm_call(zcur, gsrc, ssrc, zpat64)
    if l == N_LAYERS - 1:
      scur, _ = _scale_fin(acc, dinv, scur)
    else:
      scur, zcur = _scale_mid(acc, dinv, scur)
  return scur
